# baseline (speedup 1.0000x reference)
import functools

import jax
import jax.numpy as jnp
from jax import lax
from jax.experimental import pallas as pl
from jax.experimental.pallas import tpu as pltpu
from jax.experimental.pallas import tpu_sc as plsc

BATCH = 16384
EMBED = 32
LANES = 16


@functools.lru_cache(maxsize=None)
def _make_kernel(num_cores: int, num_subcores: int):
    nw = num_cores * num_subcores
    bpw = BATCH // nw
    groups = bpw // LANES
    mesh = plsc.VectorSubcoreMesh(
        core_axis_name="c", subcore_axis_name="s", num_cores=num_cores
    )

    @functools.partial(
        pl.kernel,
        mesh=mesh,
        compiler_params=pltpu.CompilerParams(
            needs_layout_passes=False, use_tc_tiling_on_sc=False),
        out_type=jax.ShapeDtypeStruct((BATCH,), jnp.float32),
        scratch_types=[
            pltpu.VMEM((bpw,), jnp.int32),
            pltpu.VMEM((bpw,), jnp.int32),
            pltpu.VMEM((bpw, EMBED), jnp.float32),
            pltpu.VMEM((bpw, EMBED), jnp.float32),
            pltpu.VMEM((bpw,), jnp.float32),
            pltpu.VMEM((bpw,), jnp.float32),
            pltpu.VMEM((bpw,), jnp.float32),
            pltpu.SemaphoreType.DMA,
            pltpu.SemaphoreType.DMA,
            pltpu.SemaphoreType.DMA,
            pltpu.SemaphoreType.DMA,
        ],
    )
    def funk_kernel(user_hbm, item_hbm, user_mf_hbm, item_mf_hbm,
                    user_bias_hbm, item_bias_hbm, out_hbm,
                    uidx_v, iidx_v, urows_v, irows_v, ub_v, ib_v, out_v,
                    sem_u, sem_i, sem_ub, sem_ib):
        wid = lax.axis_index("s") * num_cores + lax.axis_index("c")
        base = wid * bpw

        pltpu.sync_copy(user_hbm.at[pl.ds(base, bpw)], uidx_v)
        pltpu.sync_copy(item_hbm.at[pl.ds(base, bpw)], iidx_v)

        cp_u = pltpu.async_copy(user_mf_hbm.at[uidx_v], urows_v, sem_u)
        cp_i = pltpu.async_copy(item_mf_hbm.at[iidx_v], irows_v, sem_i)
        cp_ub = pltpu.async_copy(user_bias_hbm.at[uidx_v], ub_v, sem_ub)
        cp_ib = pltpu.async_copy(item_bias_hbm.at[iidx_v], ib_v, sem_ib)
        cp_u.wait()
        cp_i.wait()
        cp_ub.wait()
        cp_ib.wait()

        def body(g, carry):
            rows = g * LANES + lax.iota(jnp.int32, LANES)
            acc = ub_v[pl.ds(g * LANES, LANES)] + ib_v[pl.ds(g * LANES, LANES)]
            for d in range(EMBED):
                col = jnp.full((LANES,), d, jnp.int32)
                acc = acc + (plsc.load_gather(urows_v, [rows, col])
                             * plsc.load_gather(irows_v, [rows, col]))
            out_v[pl.ds(g * LANES, LANES)] = acc
            return carry

        lax.fori_loop(0, groups, body, 0)
        pltpu.sync_copy(out_v, out_hbm.at[pl.ds(base, bpw)])

    return funk_kernel


def kernel(user, item, user_mf, item_mf, user_bias, item_bias):
    info = plsc.get_sparse_core_info()
    k = _make_kernel(info.num_cores, info.num_subcores)
    return k(user.astype(jnp.int32), item.astype(jnp.int32),
             user_mf, item_mf,
             user_bias.reshape(-1), item_bias.reshape(-1))
